# baseline (speedup 1.0000x reference)
import functools

import jax
import jax.numpy as jnp
from jax import lax
from jax.experimental import pallas as pl
from jax.experimental.pallas import tpu as pltpu
from jax.experimental.pallas import tpu_sc as plsc

B = 16
T = 2048
D = 1024
LANES = 16
NGROUPS = T // LANES
IDX_CHUNK = 128
NCHUNKS = T // IDX_CHUNK
PS = D // LANES

_mesh = plsc.VectorSubcoreMesh(core_axis_name="c", subcore_axis_name="s")


@functools.partial(
    pl.kernel,
    out_type=jax.ShapeDtypeStruct((B * PS, LANES), jnp.float32),
    mesh=_mesh,
    compiler_params=pltpu.CompilerParams(
        use_tc_tiling_on_sc=False, needs_layout_passes=False
    ),
    scratch_types=[
        pltpu.VMEM((NCHUNKS, IDX_CHUNK), jnp.int32),
        pltpu.VMEM((T, LANES), jnp.float32),
        pltpu.VMEM((PS, LANES), jnp.float32),
        pltpu.VMEM((LANES,), jnp.int32),
        pltpu.SMEM((1,), jnp.int32),
        pltpu.SemaphoreType.DMA,
    ],
)
def _reduce_last_sc(flat_hbm, out_hbm, idx_v, probes_v, rowbuf_v, cnt_v,
                    total_ref, sem):
    num_cores = 2
    wid = lax.axis_index("s") * num_cores + lax.axis_index("c")

    @pl.when(wid < B)
    def _body():
        b = wid
        iota = lax.iota(jnp.int32, LANES)
        row0 = b * T

        for j in range(NCHUNKS):
            for v in range(IDX_CHUNK // LANES):
                base = (row0 + j * IDX_CHUNK + v * LANES) * PS
                idx_v[j, pl.ds(v * LANES, LANES)] = base + iota * PS

        copies = [
            pltpu.async_copy(
                flat_hbm.at[idx_v.at[j]],
                probes_v.at[pl.ds(j * IDX_CHUNK, IDX_CHUNK)],
                sem,
            )
            for j in range(NCHUNKS)
        ]
        for cp in copies:
            cp.wait()

        def group_body(g, cnt):
            base = g * LANES
            for r in range(LANES):
                vals = probes_v[base + r, :]
                pc = plsc.all_reduce_population_count(vals != 0.0)
                cnt = cnt + (pc > 0).astype(jnp.int32)
            return cnt

        cnt_vec = lax.fori_loop(0, NGROUPS, group_body,
                                jnp.zeros((LANES,), jnp.int32))
        cnt_v[...] = cnt_vec
        count_fast = cnt_v[...][0]
        total_ref[0] = count_fast

        @pl.when(count_fast < T)
        def _slow():
            def row_body(r, cnt):
                pltpu.sync_copy(
                    flat_hbm.at[pl.ds((row0 + r) * PS, PS)], rowbuf_v
                )
                acc = jnp.zeros((LANES,), jnp.int32)
                for c in range(PS):
                    seg = rowbuf_v[c, :]
                    acc = acc + (seg != 0.0).astype(jnp.int32)
                pc = plsc.all_reduce_population_count(acc > 0)
                return cnt + (pc > 0).astype(jnp.int32)

            total_vec = lax.fori_loop(0, T, row_body,
                                      jnp.zeros((LANES,), jnp.int32))
            cnt_v[...] = total_vec
            total_ref[0] = cnt_v[...][0]

        last = jnp.maximum(total_ref[0] - 1, 0)
        pltpu.sync_copy(flat_hbm.at[pl.ds((row0 + last) * PS, PS)], rowbuf_v)
        pltpu.sync_copy(rowbuf_v, out_hbm.at[pl.ds(b * PS, PS)])


def kernel(inputs):
    flat16 = inputs.reshape(B * T * D // LANES, LANES)
    return _reduce_last_sc(flat16).reshape(B, D)
